# baseline (speedup 1.0000x reference)
import functools

import jax
import jax.numpy as jnp
from jax import lax
from jax.experimental import pallas as pl
from jax.experimental.pallas import tpu as pltpu

_ST = 128
_WR = 128
_RC = 512
_NBUF = 4


def _make_kern(N):
    def _kern(x_hbm, starts_ref, ends_ref, starts_s, ends_s, wi_ref, bi_ref,
              wo_ref, bo_ref, out_ref, xbuf, cbuf, acc_ref, sem, csem):
        Bp = starts_ref.shape[0]
        T = Bp // _ST
        B = out_ref.shape[0]

        acc_ref[...] = jnp.zeros_like(acc_ref)
        io = lax.broadcasted_iota(jnp.int32, (_ST, _WR), 1)
        ioc = lax.broadcasted_iota(jnp.int32, (_ST, _RC), 1)

        def r0_of(t):
            return starts_s[t * _ST]

        def r1_of(t):
            return ends_s[t * _ST + _ST - 1]

        def win_copy(t):
            d0 = jnp.minimum(r0_of(t), N - _WR)
            return pltpu.make_async_copy(
                x_hbm.at[pl.ds(d0, _WR), :], xbuf.at[t % _NBUF],
                sem.at[t % _NBUF])

        for t in range(min(_NBUF, T)):
            @pl.when(r1_of(t) > r0_of(t))
            def _(t=t):
                win_copy(t).start()

        for t in range(T):
            @pl.when(r1_of(t) > r0_of(t))
            def _(t=t):
                win_copy(t).wait()
                d0 = jnp.minimum(r0_of(t), N - _WR)
                st = starts_ref[t * _ST:(t + 1) * _ST, :]
                en = ends_ref[t * _ST:(t + 1) * _ST, :]
                r = io + d0
                m = jnp.where((r >= st) & (r < en), 1.0, 0.0)
                acc_ref[t * _ST:(t + 1) * _ST, :] += lax.dot_general(
                    m, xbuf[t % _NBUF], (((1,), (0,)), ((), ())),
                    preferred_element_type=jnp.float32)

            if t + _NBUF < T:
                @pl.when(r1_of(t + _NBUF) > r0_of(t + _NBUF))
                def _(t=t):
                    win_copy(t + _NBUF).start()

        for t in range(T):
            extra = r1_of(t) - r0_of(t) - _WR
            trip = lax.div(jnp.maximum(extra, 0) + (_RC - 1), _RC)
            st = starts_ref[t * _ST:(t + 1) * _ST, :]
            en = ends_ref[t * _ST:(t + 1) * _ST, :]

            def body(j, carry, t=t, st=st, en=en):
                rr0 = r0_of(t) + _WR + j * _RC
                d0 = jnp.minimum(rr0, N - _RC)
                cp = pltpu.make_async_copy(
                    x_hbm.at[pl.ds(d0, _RC), :], cbuf, csem)
                cp.start()
                cp.wait()
                r = ioc + d0
                m = jnp.where((r >= st) & (r < en) & (r >= rr0), 1.0, 0.0)
                acc_ref[t * _ST:(t + 1) * _ST, :] += lax.dot_general(
                    m, cbuf[...], (((1,), (0,)), ((), ())),
                    preferred_element_type=jnp.float32)
                return carry

            lax.fori_loop(0, trip, body, 0)

        agg = acc_ref[...]
        h = lax.dot_general(agg, wi_ref[...], (((1,), (1,)), ((), ())),
                            preferred_element_type=jnp.float32) + bi_ref[...]
        sp = jnp.maximum(h, 0.0) + jnp.log1p(jnp.exp(-jnp.abs(h)))
        h = h * jnp.tanh(sp)
        y = lax.dot_general(h, wo_ref[...], (((1,), (1,)), ((), ())),
                            preferred_element_type=jnp.float32) + bo_ref[...]
        out_ref[...] = y[:B, :]

    return _kern


@functools.partial(jax.jit, static_argnames=("interpret",))
def _sum_readout(node_embeddings, node_sizes, W_inner, b_inner, W_outer,
                 b_outer, interpret=False):
    N, d_in = node_embeddings.shape
    B = node_sizes.shape[0]
    d_out = W_outer.shape[0]
    Bp = ((B + _ST - 1) // _ST) * _ST

    sizes = node_sizes.astype(jnp.int32)
    ends_i = jnp.cumsum(sizes)
    starts_i = ends_i - sizes
    n_rows = ends_i[-1]
    pad = jnp.full((Bp - B,), n_rows, jnp.int32)
    ends_p = jnp.concatenate([ends_i, pad])
    starts_p = jnp.concatenate([starts_i, pad])

    out = pl.pallas_call(
        _make_kern(N),
        out_shape=jax.ShapeDtypeStruct((B, d_out), jnp.float32),
        in_specs=[
            pl.BlockSpec(memory_space=pl.ANY),
            pl.BlockSpec(memory_space=pltpu.VMEM),
            pl.BlockSpec(memory_space=pltpu.VMEM),
            pl.BlockSpec(memory_space=pltpu.SMEM),
            pl.BlockSpec(memory_space=pltpu.SMEM),
            pl.BlockSpec(memory_space=pltpu.VMEM),
            pl.BlockSpec(memory_space=pltpu.VMEM),
            pl.BlockSpec(memory_space=pltpu.VMEM),
            pl.BlockSpec(memory_space=pltpu.VMEM),
        ],
        out_specs=pl.BlockSpec(memory_space=pltpu.VMEM),
        scratch_shapes=[
            pltpu.VMEM((_NBUF, _WR, d_in), jnp.float32),
            pltpu.VMEM((_RC, d_in), jnp.float32),
            pltpu.VMEM((Bp, d_in), jnp.float32),
            pltpu.SemaphoreType.DMA((_NBUF,)),
            pltpu.SemaphoreType.DMA,
        ],
        interpret=interpret,
    )(node_embeddings, starts_p.reshape(Bp, 1), ends_p.reshape(Bp, 1),
      starts_p, ends_p, W_inner, b_inner.reshape(1, -1), W_outer,
      b_outer.reshape(1, -1))
    return out


def kernel(node_embeddings, node_sizes, W_inner, b_inner, W_outer, b_outer):
    return _sum_readout(node_embeddings, node_sizes, W_inner, b_inner,
                        W_outer, b_outer)


# speedup vs baseline: 1.1820x; 1.1820x over previous
import functools

import jax
import jax.numpy as jnp
import numpy as np
from jax import lax
from jax.experimental import pallas as pl
from jax.experimental.pallas import tpu as pltpu

_ST = 128
_WR = 128
_RC = 512
_NBUF = 4


def _make_kern(N):
    def _kern(x_hbm, starts_ref, ends_ref, starts_s, ends_s, wi_ref, bi_ref,
              wo_ref, bo_ref, out_ref, xbuf, cbuf, acc_ref, sem, csem):
        Bp = starts_ref.shape[0]
        T = Bp // _ST
        B = out_ref.shape[0]

        acc_ref[...] = jnp.zeros_like(acc_ref)
        io = lax.broadcasted_iota(jnp.int32, (_ST, _WR), 1)
        ioc = lax.broadcasted_iota(jnp.int32, (_ST, _RC), 1)

        def r0_of(t):
            return starts_s[t * _ST]

        def r1_of(t):
            return ends_s[t * _ST + _ST - 1]

        def win_copy(t):
            d0 = jnp.minimum(r0_of(t), N - _WR)
            return pltpu.make_async_copy(
                x_hbm.at[pl.ds(d0, _WR), :], xbuf.at[t % _NBUF],
                sem.at[t % _NBUF])

        for t in range(min(_NBUF, T)):
            @pl.when(r1_of(t) > r0_of(t))
            def _(t=t):
                win_copy(t).start()

        for t in range(T):
            @pl.when(r1_of(t) > r0_of(t))
            def _(t=t):
                win_copy(t).wait()
                d0 = jnp.minimum(r0_of(t), N - _WR)
                st = starts_ref[t * _ST:(t + 1) * _ST, :]
                en = ends_ref[t * _ST:(t + 1) * _ST, :]
                r = io + d0
                m = jnp.where((r >= st) & (r < en), 1.0, 0.0)
                acc_ref[t * _ST:(t + 1) * _ST, :] += lax.dot_general(
                    m, xbuf[t % _NBUF], (((1,), (0,)), ((), ())),
                    preferred_element_type=jnp.float32)

            if t + _NBUF < T:
                @pl.when(r1_of(t + _NBUF) > r0_of(t + _NBUF))
                def _(t=t):
                    win_copy(t + _NBUF).start()

        for t in range(T):
            extra = r1_of(t) - r0_of(t) - _WR
            trip = lax.div(jnp.maximum(extra, 0) + (_RC - 1), _RC)
            st = starts_ref[t * _ST:(t + 1) * _ST, :]
            en = ends_ref[t * _ST:(t + 1) * _ST, :]

            def body(j, carry, t=t, st=st, en=en):
                rr0 = r0_of(t) + _WR + j * _RC
                d0 = jnp.minimum(rr0, N - _RC)
                cp = pltpu.make_async_copy(
                    x_hbm.at[pl.ds(d0, _RC), :], cbuf, csem)
                cp.start()
                cp.wait()
                r = ioc + d0
                m = jnp.where((r >= st) & (r < en) & (r >= rr0), 1.0, 0.0)
                acc_ref[t * _ST:(t + 1) * _ST, :] += lax.dot_general(
                    m, cbuf[...], (((1,), (0,)), ((), ())),
                    preferred_element_type=jnp.float32)
                return carry

            lax.fori_loop(0, trip, body, 0)

        agg = acc_ref[...]
        h = lax.dot_general(agg, wi_ref[...], (((1,), (1,)), ((), ())),
                            preferred_element_type=jnp.float32) + bi_ref[...]
        sp = jnp.maximum(h, 0.0) + jnp.log1p(jnp.exp(-jnp.abs(h)))
        h = h * jnp.tanh(sp)
        y = lax.dot_general(h, wo_ref[...], (((1,), (1,)), ((), ())),
                            preferred_element_type=jnp.float32) + bo_ref[...]
        out_ref[...] = y[:B, :]

    return _kern


@functools.partial(jax.jit, static_argnames=("interpret",))
def _sum_readout(node_embeddings, node_sizes, W_inner, b_inner, W_outer,
                 b_outer, interpret=False):
    N, d_in = node_embeddings.shape
    B = node_sizes.shape[0]
    d_out = W_outer.shape[0]
    Bp = ((B + _ST - 1) // _ST) * _ST

    G = Bp // 128
    sizes_f = jnp.pad(node_sizes.astype(jnp.float32),
                      (0, Bp - B)).reshape(G, 128)
    triu = np.triu(np.ones((128, 128), np.float32))
    lstrict = np.tril(np.ones((G, G), np.float32), k=-1)
    within = sizes_f @ triu
    off = lstrict @ within[:, -1:]
    ends_p = (within + off).reshape(Bp).astype(jnp.int32)
    starts_p = ends_p - sizes_f.reshape(Bp).astype(jnp.int32)

    out = pl.pallas_call(
        _make_kern(N),
        out_shape=jax.ShapeDtypeStruct((B, d_out), jnp.float32),
        in_specs=[
            pl.BlockSpec(memory_space=pl.ANY),
            pl.BlockSpec(memory_space=pltpu.VMEM),
            pl.BlockSpec(memory_space=pltpu.VMEM),
            pl.BlockSpec(memory_space=pltpu.SMEM),
            pl.BlockSpec(memory_space=pltpu.SMEM),
            pl.BlockSpec(memory_space=pltpu.VMEM),
            pl.BlockSpec(memory_space=pltpu.VMEM),
            pl.BlockSpec(memory_space=pltpu.VMEM),
            pl.BlockSpec(memory_space=pltpu.VMEM),
        ],
        out_specs=pl.BlockSpec(memory_space=pltpu.VMEM),
        scratch_shapes=[
            pltpu.VMEM((_NBUF, _WR, d_in), jnp.float32),
            pltpu.VMEM((_RC, d_in), jnp.float32),
            pltpu.VMEM((Bp, d_in), jnp.float32),
            pltpu.SemaphoreType.DMA((_NBUF,)),
            pltpu.SemaphoreType.DMA,
        ],
        interpret=interpret,
    )(node_embeddings, starts_p.reshape(Bp, 1), ends_p.reshape(Bp, 1),
      starts_p, ends_p, W_inner, b_inner.reshape(1, -1), W_outer,
      b_outer.reshape(1, -1))
    return out


def kernel(node_embeddings, node_sizes, W_inner, b_inner, W_outer, b_outer):
    return _sum_readout(node_embeddings, node_sizes, W_inner, b_inner,
                        W_outer, b_outer)


# speedup vs baseline: 1.7208x; 1.4559x over previous
import functools

import jax
import jax.numpy as jnp
from jax import lax
from jax.experimental import pallas as pl
from jax.experimental.pallas import tpu as pltpu

_ST = 128
_WR = 128
_RC = 512
_NBUF = 4


def _make_kern(N):
    def _kern(x_hbm, sizes_ref, wi_ref, bi_ref, wo_ref, bo_ref, out_ref,
              xbuf, cbuf, acc_ref, bnd_vmem, bnd_smem, sem, csem, bsem):
        G, L = sizes_ref.shape
        Bp = G * L
        T = Bp // _ST
        B = out_ref.shape[0]

        sizes = sizes_ref[...]
        ii = lax.broadcasted_iota(jnp.int32, (L, L), 0)
        jj = lax.broadcasted_iota(jnp.int32, (L, L), 1)
        triu = jnp.where(ii <= jj, 1.0, 0.0)
        ident = jnp.where(ii == jj, 1.0, 0.0)
        gi = lax.broadcasted_iota(jnp.int32, (G, G), 0)
        gj = lax.broadcasted_iota(jnp.int32, (G, G), 1)
        lstrict = jnp.where(gj < gi, 1.0, 0.0)
        within = lax.dot_general(sizes, triu, (((1,), (0,)), ((), ())),
                                 preferred_element_type=jnp.float32)
        tot = within[:, L - 1:L]
        off = lax.dot_general(lstrict, tot, (((1,), (0,)), ((), ())),
                              preferred_element_type=jnp.float32)
        ends_g = within + off
        starts_g = ends_g - sizes
        ends_t = lax.dot_general(ident, ends_g, (((1,), (1,)), ((), ())),
                                 preferred_element_type=jnp.float32)
        starts_t = lax.dot_general(ident, starts_g, (((1,), (1,)), ((), ())),
                                   preferred_element_type=jnp.float32)
        bnd_vmem[:, 0:1] = off.astype(jnp.int32)
        bnd_vmem[:, 1:2] = (off + tot).astype(jnp.int32)
        bcp = pltpu.make_async_copy(bnd_vmem, bnd_smem, bsem)
        bcp.start()
        bcp.wait()

        acc_ref[...] = jnp.zeros_like(acc_ref)
        iof = lax.broadcasted_iota(jnp.int32, (_ST, _WR), 1).astype(
            jnp.float32)
        iocf = lax.broadcasted_iota(jnp.int32, (_ST, _RC), 1).astype(
            jnp.float32)

        def r0_of(t):
            return bnd_smem[t, 0]

        def r1_of(t):
            return bnd_smem[t, 1]

        def win_copy(t):
            d0 = jnp.minimum(r0_of(t), N - _WR)
            return pltpu.make_async_copy(
                x_hbm.at[pl.ds(d0, _WR), :], xbuf.at[t % _NBUF],
                sem.at[t % _NBUF])

        for t in range(min(_NBUF, T)):
            @pl.when(r1_of(t) > r0_of(t))
            def _(t=t):
                win_copy(t).start()

        for t in range(T):
            @pl.when(r1_of(t) > r0_of(t))
            def _(t=t):
                win_copy(t).wait()
                d0 = jnp.minimum(r0_of(t), N - _WR)
                st = starts_t[:, t:t + 1]
                en = ends_t[:, t:t + 1]
                r = iof + d0.astype(jnp.float32)
                m = jnp.where((r >= st) & (r < en), 1.0, 0.0)
                acc_ref[t * _ST:(t + 1) * _ST, :] += lax.dot_general(
                    m, xbuf[t % _NBUF], (((1,), (0,)), ((), ())),
                    preferred_element_type=jnp.float32)

            if t + _NBUF < T:
                @pl.when(r1_of(t + _NBUF) > r0_of(t + _NBUF))
                def _(t=t):
                    win_copy(t + _NBUF).start()

        for t in range(T):
            extra = r1_of(t) - r0_of(t) - _WR
            trip = lax.div(jnp.maximum(extra, 0) + (_RC - 1), _RC)
            st = starts_t[:, t:t + 1]
            en = ends_t[:, t:t + 1]

            def body(j, carry, t=t, st=st, en=en):
                rr0 = r0_of(t) + _WR + j * _RC
                d0 = jnp.minimum(rr0, N - _RC)
                cp = pltpu.make_async_copy(
                    x_hbm.at[pl.ds(d0, _RC), :], cbuf, csem)
                cp.start()
                cp.wait()
                r = iocf + d0.astype(jnp.float32)
                m = jnp.where((r >= st) & (r < en)
                              & (r >= rr0.astype(jnp.float32)), 1.0, 0.0)
                acc_ref[t * _ST:(t + 1) * _ST, :] += lax.dot_general(
                    m, cbuf[...], (((1,), (0,)), ((), ())),
                    preferred_element_type=jnp.float32)
                return carry

            lax.fori_loop(0, trip, body, 0)

        agg = acc_ref[...]
        h = lax.dot_general(agg, wi_ref[...], (((1,), (1,)), ((), ())),
                            preferred_element_type=jnp.float32) + bi_ref[...]
        sp = jnp.maximum(h, 0.0) + jnp.log1p(jnp.exp(-jnp.abs(h)))
        h = h * jnp.tanh(sp)
        y = lax.dot_general(h, wo_ref[...], (((1,), (1,)), ((), ())),
                            preferred_element_type=jnp.float32) + bo_ref[...]
        out_ref[...] = y[:B, :]

    return _kern


@functools.partial(jax.jit, static_argnames=("interpret",))
def _sum_readout(node_embeddings, node_sizes, W_inner, b_inner, W_outer,
                 b_outer, interpret=False):
    N, d_in = node_embeddings.shape
    B = node_sizes.shape[0]
    d_out = W_outer.shape[0]
    Bp = ((B + 127) // 128) * 128
    G = Bp // 128

    sizes_f = jnp.pad(node_sizes.astype(jnp.float32),
                      (0, Bp - B)).reshape(G, 128)

    out = pl.pallas_call(
        _make_kern(N),
        out_shape=jax.ShapeDtypeStruct((B, d_out), jnp.float32),
        in_specs=[
            pl.BlockSpec(memory_space=pl.ANY),
            pl.BlockSpec(memory_space=pltpu.VMEM),
            pl.BlockSpec(memory_space=pltpu.VMEM),
            pl.BlockSpec(memory_space=pltpu.VMEM),
            pl.BlockSpec(memory_space=pltpu.VMEM),
            pl.BlockSpec(memory_space=pltpu.VMEM),
        ],
        out_specs=pl.BlockSpec(memory_space=pltpu.VMEM),
        scratch_shapes=[
            pltpu.VMEM((_NBUF, _WR, d_in), jnp.float32),
            pltpu.VMEM((_RC, d_in), jnp.float32),
            pltpu.VMEM((Bp, d_in), jnp.float32),
            pltpu.VMEM((G, 2), jnp.int32),
            pltpu.SMEM((G, 2), jnp.int32),
            pltpu.SemaphoreType.DMA((_NBUF,)),
            pltpu.SemaphoreType.DMA,
            pltpu.SemaphoreType.DMA,
        ],
        interpret=interpret,
    )(node_embeddings, sizes_f, W_inner, b_inner.reshape(1, -1), W_outer,
      b_outer.reshape(1, -1))
    return out


def kernel(node_embeddings, node_sizes, W_inner, b_inner, W_outer, b_outer):
    return _sum_readout(node_embeddings, node_sizes, W_inner, b_inner,
                        W_outer, b_outer)


# speedup vs baseline: 1.7293x; 1.0049x over previous
import functools

import jax
import jax.numpy as jnp
from jax import lax
from jax.experimental import pallas as pl
from jax.experimental.pallas import tpu as pltpu

_ST = 128
_WR = 128
_RC = 512
_NBUF = 4


def _make_kern(N):
    def _kern(x_hbm, sizes_ref, bnd_smem, wi_ref, bi_ref, wo_ref, bo_ref,
              out_ref, xbuf, cbuf, acc_ref, sem, csem):
        G, L = sizes_ref.shape
        Bp = G * L
        T = Bp // _ST
        B = out_ref.shape[0]

        sizes = sizes_ref[...]
        ii = lax.broadcasted_iota(jnp.int32, (L, L), 0)
        jj = lax.broadcasted_iota(jnp.int32, (L, L), 1)
        triu = jnp.where(ii <= jj, 1.0, 0.0)
        ident = jnp.where(ii == jj, 1.0, 0.0)
        gi = lax.broadcasted_iota(jnp.int32, (G, G), 0)
        gj = lax.broadcasted_iota(jnp.int32, (G, G), 1)
        lstrict = jnp.where(gj < gi, 1.0, 0.0)
        within = lax.dot_general(sizes, triu, (((1,), (0,)), ((), ())),
                                 preferred_element_type=jnp.float32)
        tot = within[:, L - 1:L]
        off = lax.dot_general(lstrict, tot, (((1,), (0,)), ((), ())),
                              preferred_element_type=jnp.float32)
        ends_g = within + off
        starts_g = ends_g - sizes
        ends_t = lax.dot_general(ident, ends_g, (((1,), (1,)), ((), ())),
                                 preferred_element_type=jnp.float32)
        starts_t = lax.dot_general(ident, starts_g, (((1,), (1,)), ((), ())),
                                   preferred_element_type=jnp.float32)
        acc_ref[...] = jnp.zeros_like(acc_ref)
        iof = lax.broadcasted_iota(jnp.int32, (_ST, _WR), 1).astype(
            jnp.float32)
        iocf = lax.broadcasted_iota(jnp.int32, (_ST, _RC), 1).astype(
            jnp.float32)

        def r0_of(t):
            return bnd_smem[t, 0]

        def r1_of(t):
            return bnd_smem[t, 1]

        def win_copy(t):
            d0 = jnp.minimum(r0_of(t), N - _WR)
            return pltpu.make_async_copy(
                x_hbm.at[pl.ds(d0, _WR), :], xbuf.at[t % _NBUF],
                sem.at[t % _NBUF])

        for t in range(min(_NBUF, T)):
            @pl.when(r1_of(t) > r0_of(t))
            def _(t=t):
                win_copy(t).start()

        for t in range(T):
            @pl.when(r1_of(t) > r0_of(t))
            def _(t=t):
                win_copy(t).wait()
                d0 = jnp.minimum(r0_of(t), N - _WR)
                st = starts_t[:, t:t + 1]
                en = ends_t[:, t:t + 1]
                r = iof + d0.astype(jnp.float32)
                m = jnp.where((r >= st) & (r < en), 1.0, 0.0)
                acc_ref[t * _ST:(t + 1) * _ST, :] += lax.dot_general(
                    m, xbuf[t % _NBUF], (((1,), (0,)), ((), ())),
                    preferred_element_type=jnp.float32)

            if t + _NBUF < T:
                @pl.when(r1_of(t + _NBUF) > r0_of(t + _NBUF))
                def _(t=t):
                    win_copy(t + _NBUF).start()

        for t in range(T):
            extra = r1_of(t) - r0_of(t) - _WR
            trip = lax.div(jnp.maximum(extra, 0) + (_RC - 1), _RC)
            st = starts_t[:, t:t + 1]
            en = ends_t[:, t:t + 1]

            def body(j, carry, t=t, st=st, en=en):
                rr0 = r0_of(t) + _WR + j * _RC
                d0 = jnp.minimum(rr0, N - _RC)
                cp = pltpu.make_async_copy(
                    x_hbm.at[pl.ds(d0, _RC), :], cbuf, csem)
                cp.start()
                cp.wait()
                r = iocf + d0.astype(jnp.float32)
                m = jnp.where((r >= st) & (r < en)
                              & (r >= rr0.astype(jnp.float32)), 1.0, 0.0)
                acc_ref[t * _ST:(t + 1) * _ST, :] += lax.dot_general(
                    m, cbuf[...], (((1,), (0,)), ((), ())),
                    preferred_element_type=jnp.float32)
                return carry

            lax.fori_loop(0, trip, body, 0)

        agg = acc_ref[...]
        h = lax.dot_general(agg, wi_ref[...], (((1,), (1,)), ((), ())),
                            preferred_element_type=jnp.float32) + bi_ref[...]
        sp = jnp.maximum(h, 0.0) + jnp.log1p(jnp.exp(-jnp.abs(h)))
        h = h * jnp.tanh(sp)
        y = lax.dot_general(h, wo_ref[...], (((1,), (1,)), ((), ())),
                            preferred_element_type=jnp.float32) + bo_ref[...]
        out_ref[...] = y[:B, :]

    return _kern


@functools.partial(jax.jit, static_argnames=("interpret",))
def _sum_readout(node_embeddings, node_sizes, W_inner, b_inner, W_outer,
                 b_outer, interpret=False):
    N, d_in = node_embeddings.shape
    B = node_sizes.shape[0]
    d_out = W_outer.shape[0]
    Bp = ((B + 127) // 128) * 128
    G = Bp // 128

    sizes_f = jnp.pad(node_sizes.astype(jnp.float32),
                      (0, Bp - B)).reshape(G, 128)
    tile_tot = jnp.sum(sizes_f, axis=1).astype(jnp.int32)
    tile_end = jnp.cumsum(tile_tot)
    bnd = jnp.stack([tile_end - tile_tot, tile_end], axis=1)

    out = pl.pallas_call(
        _make_kern(N),
        out_shape=jax.ShapeDtypeStruct((B, d_out), jnp.float32),
        in_specs=[
            pl.BlockSpec(memory_space=pl.ANY),
            pl.BlockSpec(memory_space=pltpu.VMEM),
            pl.BlockSpec(memory_space=pltpu.SMEM),
            pl.BlockSpec(memory_space=pltpu.VMEM),
            pl.BlockSpec(memory_space=pltpu.VMEM),
            pl.BlockSpec(memory_space=pltpu.VMEM),
            pl.BlockSpec(memory_space=pltpu.VMEM),
        ],
        out_specs=pl.BlockSpec(memory_space=pltpu.VMEM),
        scratch_shapes=[
            pltpu.VMEM((_NBUF, _WR, d_in), jnp.float32),
            pltpu.VMEM((_RC, d_in), jnp.float32),
            pltpu.VMEM((Bp, d_in), jnp.float32),
            pltpu.SemaphoreType.DMA((_NBUF,)),
            pltpu.SemaphoreType.DMA,
        ],
        interpret=interpret,
    )(node_embeddings, sizes_f, bnd, W_inner, b_inner.reshape(1, -1),
      W_outer, b_outer.reshape(1, -1))
    return out


def kernel(node_embeddings, node_sizes, W_inner, b_inner, W_outer, b_outer):
    return _sum_readout(node_embeddings, node_sizes, W_inner, b_inner,
                        W_outer, b_outer)


# speedup vs baseline: 1.7295x; 1.0001x over previous
import functools

import jax
import jax.numpy as jnp
from jax import lax
from jax.experimental import pallas as pl
from jax.experimental.pallas import tpu as pltpu

_ST = 128
_WR = 128
_RC = 512
_NBUF = 4


def _make_kern(N):
    def _kern(x_hbm, sizes_ref, r0s_smem, r1s_smem, wi_ref, bi_ref, wo_ref,
              bo_ref, out_ref, xbuf, cbuf, acc_ref, sem, csem):
        G, L = sizes_ref.shape
        Bp = G * L
        T = Bp // _ST
        B = out_ref.shape[0]

        sizes = sizes_ref[...]
        ii = lax.broadcasted_iota(jnp.int32, (L, L), 0)
        jj = lax.broadcasted_iota(jnp.int32, (L, L), 1)
        triu = jnp.where(ii <= jj, 1.0, 0.0)
        ident = jnp.where(ii == jj, 1.0, 0.0)
        gi = lax.broadcasted_iota(jnp.int32, (G, G), 0)
        gj = lax.broadcasted_iota(jnp.int32, (G, G), 1)
        lstrict = jnp.where(gj < gi, 1.0, 0.0)
        within = lax.dot_general(sizes, triu, (((1,), (0,)), ((), ())),
                                 precision=lax.Precision.HIGHEST,
                                 preferred_element_type=jnp.float32)
        tot = within[:, L - 1:L]
        off = lax.dot_general(lstrict, tot, (((1,), (0,)), ((), ())),
                              precision=lax.Precision.HIGHEST,
                              preferred_element_type=jnp.float32)
        ends_g = within + off
        starts_g = ends_g - sizes
        ends_t = lax.dot_general(ident, ends_g, (((1,), (1,)), ((), ())),
                                 precision=lax.Precision.HIGHEST,
                                 preferred_element_type=jnp.float32)
        starts_t = lax.dot_general(ident, starts_g, (((1,), (1,)), ((), ())),
                                     precision=lax.Precision.HIGHEST,
                                     preferred_element_type=jnp.float32)
        ends_ti = ends_t.astype(jnp.int32)
        starts_ti = starts_t.astype(jnp.int32)

        acc_ref[...] = jnp.zeros_like(acc_ref)
        io = lax.broadcasted_iota(jnp.int32, (_ST, _WR), 1)
        ioc = lax.broadcasted_iota(jnp.int32, (_ST, _RC), 1)

        def r0_of(t):
            return r0s_smem[t]

        def r1_of(t):
            return r1s_smem[t]

        def win_copy(t):
            d0 = jnp.minimum(r0_of(t), N - _WR)
            return pltpu.make_async_copy(
                x_hbm.at[pl.ds(d0, _WR), :], xbuf.at[t % _NBUF],
                sem.at[t % _NBUF])

        for t in range(min(_NBUF, T)):
            @pl.when(r1_of(t) > r0_of(t))
            def _(t=t):
                win_copy(t).start()

        for t in range(T):
            @pl.when(r1_of(t) > r0_of(t))
            def _(t=t):
                win_copy(t).wait()
                d0 = jnp.minimum(r0_of(t), N - _WR)
                st = starts_ti[:, t:t + 1]
                en = ends_ti[:, t:t + 1]
                r = io + d0
                m = jnp.where((r >= st) & (r < en), 1.0, 0.0)
                acc_ref[t * _ST:(t + 1) * _ST, :] += lax.dot_general(
                    m, xbuf[t % _NBUF], (((1,), (0,)), ((), ())),
                    preferred_element_type=jnp.float32)

            if t + _NBUF < T:
                @pl.when(r1_of(t + _NBUF) > r0_of(t + _NBUF))
                def _(t=t):
                    win_copy(t + _NBUF).start()

        for t in range(T):
            extra = r1_of(t) - r0_of(t) - _WR
            trip = lax.div(jnp.maximum(extra, 0) + (_RC - 1), _RC)
            st = starts_ti[:, t:t + 1]
            en = ends_ti[:, t:t + 1]

            def body(j, carry, t=t, st=st, en=en):
                rr0 = r0_of(t) + _WR + j * _RC
                d0 = jnp.minimum(rr0, N - _RC)
                cp = pltpu.make_async_copy(
                    x_hbm.at[pl.ds(d0, _RC), :], cbuf, csem)
                cp.start()
                cp.wait()
                r = ioc + d0
                m = jnp.where((r >= st) & (r < en) & (r >= rr0), 1.0, 0.0)
                acc_ref[t * _ST:(t + 1) * _ST, :] += lax.dot_general(
                    m, cbuf[...], (((1,), (0,)), ((), ())),
                    preferred_element_type=jnp.float32)
                return carry

            lax.fori_loop(0, trip, body, 0)

        agg = acc_ref[...]
        h = lax.dot_general(agg, wi_ref[...], (((1,), (1,)), ((), ())),
                            preferred_element_type=jnp.float32) + bi_ref[...]
        sp = jnp.maximum(h, 0.0) + jnp.log1p(jnp.exp(-jnp.abs(h)))
        h = h * jnp.tanh(sp)
        y = lax.dot_general(h, wo_ref[...], (((1,), (1,)), ((), ())),
                            preferred_element_type=jnp.float32) + bo_ref[...]
        out_ref[...] = y[:B, :]

    return _kern


@functools.partial(jax.jit, static_argnames=("interpret",))
def _sum_readout(node_embeddings, node_sizes, W_inner, b_inner, W_outer,
                 b_outer, interpret=False):
    N, d_in = node_embeddings.shape
    B = node_sizes.shape[0]
    d_out = W_outer.shape[0]
    Bp = ((B + 127) // 128) * 128
    G = Bp // 128

    sizes_f = jnp.pad(node_sizes.astype(jnp.float32),
                      (0, Bp - B)).reshape(G, 128)
    tile_tot = jnp.sum(sizes_f, axis=1).astype(jnp.int32)
    tile_end = jnp.cumsum(tile_tot)
    tile_start = tile_end - tile_tot

    out = pl.pallas_call(
        _make_kern(N),
        out_shape=jax.ShapeDtypeStruct((B, d_out), jnp.float32),
        in_specs=[
            pl.BlockSpec(memory_space=pl.ANY),
            pl.BlockSpec(memory_space=pltpu.VMEM),
            pl.BlockSpec(memory_space=pltpu.SMEM),
            pl.BlockSpec(memory_space=pltpu.SMEM),
            pl.BlockSpec(memory_space=pltpu.VMEM),
            pl.BlockSpec(memory_space=pltpu.VMEM),
            pl.BlockSpec(memory_space=pltpu.VMEM),
            pl.BlockSpec(memory_space=pltpu.VMEM),
        ],
        out_specs=pl.BlockSpec(memory_space=pltpu.VMEM),
        scratch_shapes=[
            pltpu.VMEM((_NBUF, _WR, d_in), jnp.float32),
            pltpu.VMEM((_RC, d_in), jnp.float32),
            pltpu.VMEM((Bp, d_in), jnp.float32),
            pltpu.SemaphoreType.DMA((_NBUF,)),
            pltpu.SemaphoreType.DMA,
        ],
        interpret=interpret,
    )(node_embeddings, sizes_f, tile_start, tile_end, W_inner,
      b_inner.reshape(1, -1), W_outer, b_outer.reshape(1, -1))
    return out


def kernel(node_embeddings, node_sizes, W_inner, b_inner, W_outer, b_outer):
    return _sum_readout(node_embeddings, node_sizes, W_inner, b_inner,
                        W_outer, b_outer)


# speedup vs baseline: 1.7800x; 1.0292x over previous
import functools

import jax
import jax.numpy as jnp
from jax import lax
from jax.experimental import pallas as pl
from jax.experimental.pallas import tpu as pltpu

_ST = 128
_WR = 128
_RC = 512
_NBUF = 4


def _make_kern(N):
    def _kern(x_hbm, sizes_ref, r0s_smem, r1s_smem, wi_ref, bi_ref, wo_ref,
              bo_ref, out_ref, xbuf, cbuf, acc_ref, sem, csem):
        G, L = sizes_ref.shape
        Bp = G * L
        T = Bp // _ST
        B = out_ref.shape[0]

        sizes = sizes_ref[...]
        ii = lax.broadcasted_iota(jnp.int32, (L, L), 0)
        jj = lax.broadcasted_iota(jnp.int32, (L, L), 1)
        triu = jnp.where(ii <= jj, 1.0, 0.0)
        ident = jnp.where(ii == jj, 1.0, 0.0)
        gi = lax.broadcasted_iota(jnp.int32, (G, G), 0)
        gj = lax.broadcasted_iota(jnp.int32, (G, G), 1)
        lstrict = jnp.where(gj < gi, 1.0, 0.0)
        within = lax.dot_general(sizes, triu, (((1,), (0,)), ((), ())),
                                 precision=lax.Precision.HIGHEST,
                                 preferred_element_type=jnp.float32)
        tot = within[:, L - 1:L]
        off = lax.dot_general(lstrict, tot, (((1,), (0,)), ((), ())),
                              precision=lax.Precision.HIGHEST,
                              preferred_element_type=jnp.float32)
        ends_g = within + off
        starts_g = ends_g - sizes
        ends_t = lax.dot_general(ident, ends_g, (((1,), (1,)), ((), ())),
                                 precision=lax.Precision.HIGHEST,
                                 preferred_element_type=jnp.float32)
        starts_t = lax.dot_general(ident, starts_g, (((1,), (1,)), ((), ())),
                                     precision=lax.Precision.HIGHEST,
                                     preferred_element_type=jnp.float32)
        ends_ti = ends_t.astype(jnp.int32)
        starts_ti = starts_t.astype(jnp.int32)

        acc_ref[...] = jnp.zeros_like(acc_ref)
        io = lax.broadcasted_iota(jnp.int32, (_ST, _WR), 1)
        ioc = lax.broadcasted_iota(jnp.int32, (_ST, _RC), 1)

        def r0_of(t):
            return r0s_smem[t]

        def r1_of(t):
            return r1s_smem[t]

        def win_copy(t):
            d0 = jnp.minimum(r0_of(t), N - _WR)
            return pltpu.make_async_copy(
                x_hbm.at[pl.ds(d0, _WR), :], xbuf.at[t % _NBUF],
                sem.at[t % _NBUF])

        for t in range(min(_NBUF, T)):
            @pl.when(r1_of(t) > r0_of(t))
            def _(t=t):
                win_copy(t).start()

        for t in range(T):
            @pl.when(r1_of(t) > r0_of(t))
            def _(t=t):
                win_copy(t).wait()
                d0 = jnp.minimum(r0_of(t), N - _WR)
                st = starts_ti[:, t:t + 1]
                en = ends_ti[:, t:t + 1]
                r = io + d0
                m = jnp.where((r >= st) & (r < en), 1.0, 0.0)
                acc_ref[t * _ST:(t + 1) * _ST, :] += lax.dot_general(
                    m, xbuf[t % _NBUF], (((1,), (0,)), ((), ())),
                    preferred_element_type=jnp.float32)

            if t + _NBUF < T:
                @pl.when(r1_of(t + _NBUF) > r0_of(t + _NBUF))
                def _(t=t):
                    win_copy(t + _NBUF).start()

        for t in range(T):
            extra = r1_of(t) - r0_of(t) - _WR
            trip = lax.div(jnp.maximum(extra, 0) + (_RC - 1), _RC)
            st = starts_ti[:, t:t + 1]
            en = ends_ti[:, t:t + 1]

            def body(j, carry, t=t, st=st, en=en):
                rr0 = r0_of(t) + _WR + j * _RC
                d0 = jnp.minimum(rr0, N - _RC)
                cp = pltpu.make_async_copy(
                    x_hbm.at[pl.ds(d0, _RC), :], cbuf, csem)
                cp.start()
                cp.wait()
                r = ioc + d0
                m = jnp.where((r >= st) & (r < en) & (r >= rr0), 1.0, 0.0)
                acc_ref[t * _ST:(t + 1) * _ST, :] += lax.dot_general(
                    m, cbuf[...], (((1,), (0,)), ((), ())),
                    preferred_element_type=jnp.float32)
                return carry

            lax.fori_loop(0, trip, body, 0)

        agg = acc_ref[...]
        h = lax.dot_general(agg, wi_ref[...], (((1,), (1,)), ((), ())),
                            preferred_element_type=jnp.float32) + bi_ref[...]
        u = jnp.exp(h)
        v = u * (u + 2.0)
        h = jnp.where(h > 20.0, h, h * (v / (v + 2.0)))
        y = lax.dot_general(h, wo_ref[...], (((1,), (1,)), ((), ())),
                            preferred_element_type=jnp.float32) + bo_ref[...]
        out_ref[...] = y[:B, :]

    return _kern


@functools.partial(jax.jit, static_argnames=("interpret",))
def _sum_readout(node_embeddings, node_sizes, W_inner, b_inner, W_outer,
                 b_outer, interpret=False):
    N, d_in = node_embeddings.shape
    B = node_sizes.shape[0]
    d_out = W_outer.shape[0]
    Bp = ((B + 127) // 128) * 128
    G = Bp // 128

    sizes_f = jnp.pad(node_sizes.astype(jnp.float32),
                      (0, Bp - B)).reshape(G, 128)
    tile_tot = jnp.sum(sizes_f, axis=1).astype(jnp.int32)
    tile_end = jnp.cumsum(tile_tot)
    tile_start = tile_end - tile_tot

    out = pl.pallas_call(
        _make_kern(N),
        out_shape=jax.ShapeDtypeStruct((B, d_out), jnp.float32),
        in_specs=[
            pl.BlockSpec(memory_space=pl.ANY),
            pl.BlockSpec(memory_space=pltpu.VMEM),
            pl.BlockSpec(memory_space=pltpu.SMEM),
            pl.BlockSpec(memory_space=pltpu.SMEM),
            pl.BlockSpec(memory_space=pltpu.VMEM),
            pl.BlockSpec(memory_space=pltpu.VMEM),
            pl.BlockSpec(memory_space=pltpu.VMEM),
            pl.BlockSpec(memory_space=pltpu.VMEM),
        ],
        out_specs=pl.BlockSpec(memory_space=pltpu.VMEM),
        scratch_shapes=[
            pltpu.VMEM((_NBUF, _WR, d_in), jnp.float32),
            pltpu.VMEM((_RC, d_in), jnp.float32),
            pltpu.VMEM((Bp, d_in), jnp.float32),
            pltpu.SemaphoreType.DMA((_NBUF,)),
            pltpu.SemaphoreType.DMA,
        ],
        interpret=interpret,
    )(node_embeddings, sizes_f, tile_start, tile_end, W_inner,
      b_inner.reshape(1, -1), W_outer, b_outer.reshape(1, -1))
    return out


def kernel(node_embeddings, node_sizes, W_inner, b_inner, W_outer, b_outer):
    return _sum_readout(node_embeddings, node_sizes, W_inner, b_inner,
                        W_outer, b_outer)
